# baseline (speedup 1.0000x reference)
import functools

import jax
import jax.numpy as jnp
from jax import lax
from jax.experimental import pallas as pl
from jax.experimental.pallas import tpu as pltpu
from jax.experimental.pallas import tpu_sc as plsc

_LANES = 16
_NUM_WORKERS = 32
_NBUF = 2


def _sc_embed(word_emb, pos_emb, ids, pids, seq, batch, hidden, chunk):
    s_span = seq * batch // _NUM_WORKERS
    n_blocks = _NUM_WORKERS // batch
    n_chunks = s_span // chunk
    vregs = chunk * hidden // _LANES
    vregs_per_row = hidden // _LANES

    mesh = plsc.VectorSubcoreMesh(core_axis_name="c", subcore_axis_name="s")

    scratch = [
        pltpu.VMEM((s_span,), jnp.int32),
        pltpu.VMEM((s_span,), jnp.int32),
    ]
    for _ in range(_NBUF):
        scratch += [
            pltpu.VMEM((chunk, hidden), jnp.float32),
            pltpu.VMEM((chunk, hidden), jnp.float32),
            pltpu.VMEM((chunk, hidden), jnp.float32),
            pltpu.SemaphoreType.DMA,
            pltpu.SemaphoreType.DMA,
        ]

    @functools.partial(
        pl.kernel,
        out_type=jax.ShapeDtypeStruct((seq, batch, hidden), jnp.float32),
        mesh=mesh,
        scratch_types=scratch,
    )
    def body(word_hbm, pos_hbm, ids_hbm, pids_hbm, out_hbm,
             idw_all, idp_all, *bufs):
        sets = [bufs[i * 5:(i + 1) * 5] for i in range(_NBUF)]
        wid = lax.axis_index("s") * 2 + lax.axis_index("c")
        b = wid % batch
        s0w = (wid // batch) * s_span

        pltpu.sync_copy(ids_hbm.at[b, pl.ds(s0w, s_span)], idw_all)
        pltpu.sync_copy(pids_hbm.at[b, pl.ds(s0w, s_span)], idp_all)

        def issue_gather(bb, g):
            wbuf, pbuf, _, gsem, _ = sets[bb]
            off = g * chunk
            pltpu.async_copy(word_hbm.at[idw_all.at[pl.ds(off, chunk)]],
                             wbuf, gsem)
            pltpu.async_copy(pos_hbm.at[idp_all.at[pl.ds(off, chunk)]],
                             pbuf, gsem)

        def wait_gather(bb, g):
            wbuf, pbuf, _, gsem, _ = sets[bb]
            off = g * chunk
            pltpu.make_async_copy(word_hbm.at[idw_all.at[pl.ds(off, chunk)]],
                                  wbuf, gsem).wait()
            pltpu.make_async_copy(pos_hbm.at[idp_all.at[pl.ds(off, chunk)]],
                                  pbuf, gsem).wait()

        def issue_out(bb, g):
            _, _, obuf, _, osem = sets[bb]
            s_base = s0w + g * chunk
            pltpu.async_copy(obuf, out_hbm.at[pl.ds(s_base, chunk), b], osem)

        def wait_out(bb, g):
            _, _, obuf, _, osem = sets[bb]
            s_base = s0w + g * chunk
            pltpu.make_async_copy(obuf, out_hbm.at[pl.ds(s_base, chunk), b],
                                  osem).wait()

        def compute(bb):
            wbuf, pbuf, obuf, _, _ = sets[bb]

            def add_body(i):
                r = i // vregs_per_row
                sl = pl.ds((i % vregs_per_row) * _LANES, _LANES)
                obuf[r, sl] = wbuf[r, sl] + pbuf[r, sl]

            plsc.parallel_loop(0, vregs, 1, unroll=16)(add_body)

        issue_gather(0, 0)
        issue_gather(1, 1)

        def outer(t, _):
            for bb in range(_NBUF):
                g = t * _NBUF + bb
                wait_gather(bb, g)

                @pl.when(g >= _NBUF)
                def _():
                    wait_out(bb, g - _NBUF)

                compute(bb)

                @pl.when(g + _NBUF < n_chunks)
                def _():
                    issue_gather(bb, g + _NBUF)

                issue_out(bb, g)
            return 0

        lax.fori_loop(0, n_chunks // _NBUF, outer, 0)
        wait_out(0, n_chunks - 2)
        wait_out(1, n_chunks - 1)

    return body(word_emb, pos_emb, ids, pids)


def kernel(input_ids, position_ids, word_emb, pos_emb):
    batch, seq = input_ids.shape
    hidden = word_emb.shape[1]

    out = _sc_embed(word_emb, pos_emb,
                    input_ids.astype(jnp.int32),
                    position_ids.astype(jnp.int32),
                    seq, batch, hidden, chunk=16)
    return out


# speedup vs baseline: 1.0187x; 1.0187x over previous
import functools

import jax
import jax.numpy as jnp
from jax import lax
from jax.experimental import pallas as pl
from jax.experimental.pallas import tpu as pltpu
from jax.experimental.pallas import tpu_sc as plsc

_LANES = 16
_NUM_WORKERS = 32
_NBUF = 4


def _sc_embed(word_emb, pos_emb, ids, pids, seq, batch, hidden, chunk):
    s_span = seq * batch // _NUM_WORKERS
    n_blocks = _NUM_WORKERS // batch
    n_chunks = s_span // chunk
    vregs = chunk * hidden // _LANES
    vregs_per_row = hidden // _LANES

    mesh = plsc.VectorSubcoreMesh(core_axis_name="c", subcore_axis_name="s")

    scratch = [
        pltpu.VMEM((s_span,), jnp.int32),
        pltpu.VMEM((s_span,), jnp.int32),
    ]
    for _ in range(_NBUF):
        scratch += [
            pltpu.VMEM((chunk, hidden), jnp.float32),
            pltpu.VMEM((chunk, hidden), jnp.float32),
            pltpu.VMEM((chunk, hidden), jnp.float32),
            pltpu.SemaphoreType.DMA,
            pltpu.SemaphoreType.DMA,
        ]

    @functools.partial(
        pl.kernel,
        out_type=jax.ShapeDtypeStruct((seq, batch, hidden), jnp.float32),
        mesh=mesh,
        scratch_types=scratch,
    )
    def body(word_hbm, pos_hbm, ids_hbm, pids_hbm, out_hbm,
             idw_all, idp_all, *bufs):
        sets = [bufs[i * 5:(i + 1) * 5] for i in range(_NBUF)]
        wid = lax.axis_index("s") * 2 + lax.axis_index("c")
        b = wid % batch
        s0w = (wid // batch) * s_span

        pltpu.sync_copy(ids_hbm.at[b, pl.ds(s0w, s_span)], idw_all)
        pltpu.sync_copy(pids_hbm.at[b, pl.ds(s0w, s_span)], idp_all)

        def issue_gather(bb, g):
            wbuf, pbuf, _, gsem, _ = sets[bb]
            off = g * chunk
            pltpu.async_copy(word_hbm.at[idw_all.at[pl.ds(off, chunk)]],
                             wbuf, gsem)
            pltpu.async_copy(pos_hbm.at[idp_all.at[pl.ds(off, chunk)]],
                             pbuf, gsem)

        def wait_gather(bb, g):
            wbuf, pbuf, _, gsem, _ = sets[bb]
            off = g * chunk
            pltpu.make_async_copy(word_hbm.at[idw_all.at[pl.ds(off, chunk)]],
                                  wbuf, gsem).wait()
            pltpu.make_async_copy(pos_hbm.at[idp_all.at[pl.ds(off, chunk)]],
                                  pbuf, gsem).wait()

        def issue_out(bb, g):
            _, _, obuf, _, osem = sets[bb]
            s_base = s0w + g * chunk
            pltpu.async_copy(obuf, out_hbm.at[pl.ds(s_base, chunk), b], osem)

        def wait_out(bb, g):
            _, _, obuf, _, osem = sets[bb]
            s_base = s0w + g * chunk
            pltpu.make_async_copy(obuf, out_hbm.at[pl.ds(s_base, chunk), b],
                                  osem).wait()

        def compute(bb):
            wbuf, pbuf, obuf, _, _ = sets[bb]

            def add_body(i):
                r = i // vregs_per_row
                sl = pl.ds((i % vregs_per_row) * _LANES, _LANES)
                obuf[r, sl] = wbuf[r, sl] + pbuf[r, sl]

            plsc.parallel_loop(0, vregs, 1, unroll=16)(add_body)

        for bb in range(_NBUF):
            issue_gather(bb, bb)

        def outer(t, _):
            for bb in range(_NBUF):
                g = t * _NBUF + bb
                wait_gather(bb, g)

                @pl.when(g >= _NBUF)
                def _():
                    wait_out(bb, g - _NBUF)

                compute(bb)

                @pl.when(g + _NBUF < n_chunks)
                def _():
                    issue_gather(bb, g + _NBUF)

                issue_out(bb, g)
            return 0

        lax.fori_loop(0, n_chunks // _NBUF, outer, 0)
        for bb in range(_NBUF):
            wait_out(bb, n_chunks - _NBUF + bb)

    return body(word_emb, pos_emb, ids, pids)


def kernel(input_ids, position_ids, word_emb, pos_emb):
    batch, seq = input_ids.shape
    hidden = word_emb.shape[1]

    out = _sc_embed(word_emb, pos_emb,
                    input_ids.astype(jnp.int32),
                    position_ids.astype(jnp.int32),
                    seq, batch, hidden, chunk=8)
    return out


# speedup vs baseline: 1.0207x; 1.0020x over previous
import functools

import jax
import jax.numpy as jnp
from jax import lax
from jax.experimental import pallas as pl
from jax.experimental.pallas import tpu as pltpu
from jax.experimental.pallas import tpu_sc as plsc

_LANES = 16
_NUM_WORKERS = 32
_NBUF = 4


def _sc_embed(word_emb, pos_emb, ids, pids, seq, batch, hidden, chunk):
    s_span = seq * batch // _NUM_WORKERS
    n_chunks = s_span // chunk
    vregs = chunk * hidden // _LANES
    vregs_per_row = hidden // _LANES

    mesh = plsc.VectorSubcoreMesh(core_axis_name="c", subcore_axis_name="s")

    scratch = [
        pltpu.VMEM((s_span,), jnp.int32),
        pltpu.VMEM((s_span,), jnp.int32),
    ]
    for _ in range(_NBUF):
        scratch += [
            pltpu.VMEM((chunk, hidden), jnp.float32),
            pltpu.VMEM((chunk, hidden), jnp.float32),
            pltpu.VMEM((chunk, hidden), jnp.float32),
            pltpu.SemaphoreType.DMA,
            pltpu.SemaphoreType.DMA,
        ]

    @functools.partial(
        pl.kernel,
        out_type=jax.ShapeDtypeStruct((seq, batch, hidden), jnp.float32),
        mesh=mesh,
        scratch_types=scratch,
    )
    def body(word_hbm, pos_hbm, ids_hbm, pids_hbm, out_hbm,
             idw_all, idp_all, *bufs):
        sets = [bufs[i * 5:(i + 1) * 5] for i in range(_NBUF)]
        wid = lax.axis_index("s") * 2 + lax.axis_index("c")
        b = wid % batch
        s0w = (wid // batch) * s_span

        pltpu.sync_copy(ids_hbm.at[b, pl.ds(s0w, s_span)], idw_all)
        pltpu.sync_copy(pids_hbm.at[b, pl.ds(s0w, s_span)], idp_all)

        def issue_gather(bb, g):
            wbuf, pbuf, _, gsem, _ = sets[bb]
            off = g * chunk
            pltpu.async_copy(word_hbm.at[idw_all.at[pl.ds(off, chunk)]],
                             wbuf, gsem)
            pltpu.async_copy(pos_hbm.at[idp_all.at[pl.ds(off, chunk)]],
                             pbuf, gsem)

        def wait_gather(bb, g):
            wbuf, pbuf, _, gsem, _ = sets[bb]
            off = g * chunk
            pltpu.make_async_copy(word_hbm.at[idw_all.at[pl.ds(off, chunk)]],
                                  wbuf, gsem).wait()
            pltpu.make_async_copy(pos_hbm.at[idp_all.at[pl.ds(off, chunk)]],
                                  pbuf, gsem).wait()

        def issue_out(bb, g):
            _, _, obuf, _, osem = sets[bb]
            s_base = s0w + g * chunk
            pltpu.async_copy(obuf, out_hbm.at[pl.ds(s_base, chunk), b], osem)

        def wait_out(bb, g):
            _, _, obuf, _, osem = sets[bb]
            s_base = s0w + g * chunk
            pltpu.make_async_copy(obuf, out_hbm.at[pl.ds(s_base, chunk), b],
                                  osem).wait()

        def compute(bb):
            wbuf, pbuf, obuf, _, _ = sets[bb]

            def add_body(i):
                r = i // vregs_per_row
                sl = pl.ds((i % vregs_per_row) * _LANES, _LANES)
                obuf[r, sl] = wbuf[r, sl] + pbuf[r, sl]

            plsc.parallel_loop(0, vregs, 1, unroll=16)(add_body)

        for bb in range(_NBUF):
            issue_gather(bb, bb)

        def outer(t, _):
            for bb in range(_NBUF):
                g = t * _NBUF + bb
                wait_gather(bb, g)

                @pl.when(g >= _NBUF)
                def _():
                    wait_out(bb, g - _NBUF)

                compute(bb)

                @pl.when(g + _NBUF < n_chunks)
                def _():
                    issue_gather(bb, g + _NBUF)

                issue_out(bb, g)
            return 0

        lax.fori_loop(0, n_chunks // _NBUF, outer, 0)
        for bb in range(_NBUF):
            wait_out(bb, n_chunks - _NBUF + bb)

    return body(word_emb, pos_emb, ids, pids)


def kernel(input_ids, position_ids, word_emb, pos_emb):
    batch, seq = input_ids.shape
    hidden = word_emb.shape[1]

    out = _sc_embed(word_emb, pos_emb,
                    input_ids.astype(jnp.int32),
                    position_ids.astype(jnp.int32),
                    seq, batch, hidden, chunk=8)
    return out
